# baseline (speedup 1.0000x reference)
import jax
import jax.numpy as jnp
from jax.experimental import pallas as pl
from jax.experimental.pallas import tpu as pltpu

_ROWS = 16 * 2048
_COLS = 1024
_NCHUNK = 4


def _copy_body(x_ref, len_ref, x_out_ref, len_out_ref, *sems):
    rows_per = _ROWS // _NCHUNK
    copies = [
        pltpu.make_async_copy(
            x_ref.at[pl.ds(i * rows_per, rows_per), :],
            x_out_ref.at[pl.ds(i * rows_per, rows_per), :],
            sems[i],
        )
        for i in range(_NCHUNK)
    ]
    len_copy = pltpu.make_async_copy(len_ref, len_out_ref, sems[_NCHUNK])
    for c in copies:
        c.start()
    len_copy.start()
    for c in copies:
        c.wait()
    len_copy.wait()


def kernel(x, x_len):
    x2 = x.reshape(_ROWS, _COLS)
    len2 = x_len.reshape(1, 16)
    out_x, out_len = pl.pallas_call(
        _copy_body,
        in_specs=[
            pl.BlockSpec(memory_space=pltpu.MemorySpace.HBM),
            pl.BlockSpec(memory_space=pltpu.MemorySpace.HBM),
        ],
        out_specs=[
            pl.BlockSpec(memory_space=pltpu.MemorySpace.HBM),
            pl.BlockSpec(memory_space=pltpu.MemorySpace.HBM),
        ],
        out_shape=[
            jax.ShapeDtypeStruct((_ROWS, _COLS), x.dtype),
            jax.ShapeDtypeStruct((1, 16), x_len.dtype),
        ],
        scratch_shapes=[pltpu.SemaphoreType.DMA] * (_NCHUNK + 1),
    )(x2, len2)
    return out_x.reshape(x.shape), out_len.reshape(x_len.shape)


# speedup vs baseline: 34.7274x; 34.7274x over previous
import functools

import jax
import jax.numpy as jnp
from jax import lax
from jax.experimental import pallas as pl
from jax.experimental.pallas import tpu as pltpu
from jax.experimental.pallas import tpu_sc as plsc

_ROWS = 16 * 2048
_COLS = 1024
_NC = 2
_NS = 16
_NW = _NC * _NS
_RPW = _ROWS // _NW
_CHUNK = 32
_NCHUNKS = _RPW // _CHUNK


def _sc_body(x_hbm, len_hbm, x_out, len_out,
             buf0, buf1, len_buf, rsem0, rsem1, wsem0, wsem1):
    c = lax.axis_index("c")
    s = lax.axis_index("s")
    wid = s * _NC + c
    base = wid * _RPW

    bufs = (buf0, buf1)
    rsems = (rsem0, rsem1)
    wsems = (wsem0, wsem1)

    writes = [None] * _NCHUNKS
    for i in range(_NCHUNKS):
        b = i % 2
        if i >= 2:
            writes[i - 2].wait()
        rd = pltpu.make_async_copy(
            x_hbm.at[pl.ds(base + i * _CHUNK, _CHUNK), :], bufs[b], rsems[b])
        rd.start()
        rd.wait()
        wr = pltpu.make_async_copy(
            bufs[b], x_out.at[pl.ds(base + i * _CHUNK, _CHUNK), :], wsems[b])
        wr.start()
        writes[i] = wr
    writes[_NCHUNKS - 2].wait()
    writes[_NCHUNKS - 1].wait()

    @pl.when(wid == 0)
    def _():
        pltpu.sync_copy(len_hbm, len_buf)
        pltpu.sync_copy(len_buf, len_out)


@functools.partial(
    pl.kernel,
    out_type=[
        jax.ShapeDtypeStruct((_ROWS, _COLS), jnp.float32),
        jax.ShapeDtypeStruct((16,), jnp.int32),
    ],
    mesh=plsc.VectorSubcoreMesh(core_axis_name="c", subcore_axis_name="s"),
    scratch_types=[
        pltpu.VMEM((_CHUNK, _COLS), jnp.float32),
        pltpu.VMEM((_CHUNK, _COLS), jnp.float32),
        pltpu.VMEM((16,), jnp.int32),
        pltpu.SemaphoreType.DMA,
        pltpu.SemaphoreType.DMA,
        pltpu.SemaphoreType.DMA,
        pltpu.SemaphoreType.DMA,
    ],
)
def _sc_copy(x_hbm, len_hbm, x_out, len_out, *scratch):
    _sc_body(x_hbm, len_hbm, x_out, len_out, *scratch)


def kernel(x, x_len):
    x2 = x.reshape(_ROWS, _COLS)
    out_x, out_len = _sc_copy(x2, x_len)
    return out_x.reshape(x.shape), out_len


# speedup vs baseline: 48.6406x; 1.4006x over previous
import jax
import jax.numpy as jnp
from jax.experimental import pallas as pl
from jax.experimental.pallas import tpu as pltpu

_ROWS = 16 * 2048
_COLS = 1024
_BLOCK_ROWS = 2048


def _copy_body(x_ref, len_ref, x_out_ref, len_out_ref):
    x_out_ref[...] = x_ref[...]
    len_out_ref[...] = len_ref[...]


def kernel(x, x_len):
    x2 = x.reshape(_ROWS, _COLS)
    len2 = x_len.reshape(1, 16)
    out_x, out_len = pl.pallas_call(
        _copy_body,
        grid=(_ROWS // _BLOCK_ROWS,),
        in_specs=[
            pl.BlockSpec((_BLOCK_ROWS, _COLS), lambda i: (i, 0)),
            pl.BlockSpec((1, 16), lambda i: (0, 0)),
        ],
        out_specs=[
            pl.BlockSpec((_BLOCK_ROWS, _COLS), lambda i: (i, 0)),
            pl.BlockSpec((1, 16), lambda i: (0, 0)),
        ],
        out_shape=[
            jax.ShapeDtypeStruct((_ROWS, _COLS), x.dtype),
            jax.ShapeDtypeStruct((1, 16), x_len.dtype),
        ],
        compiler_params=pltpu.CompilerParams(
            dimension_semantics=("arbitrary",),
        ),
    )(x2, len2)
    return out_x.reshape(x.shape), out_len.reshape(x_len.shape)
